# baseline (speedup 1.0000x reference)
import functools
import jax
import jax.numpy as jnp
import numpy as np
from jax import lax
from jax.experimental import pallas as pl
from jax.experimental.pallas import tpu as pltpu
from jax.experimental.pallas import tpu_sc as plsc

EPS = 1e-5
NW = 32

_PACK_PERM = np.stack([np.arange(16), np.arange(16) + 16], axis=1).reshape(32)


def _table_body(p_ref, f_ref, wd_ref, wf_ref, bf_ref, o_ref):
    u = jnp.dot(p_ref[...], wd_ref[...], preferred_element_type=jnp.float32)
    v = jnp.dot(f_ref[...], wf_ref[...], preferred_element_type=jnp.float32)
    v = v + bf_ref[...]
    o_ref[...] = jnp.concatenate([u, v], axis=1)


def _build_table(ptsT, featsT, wdT, wfT, bf):
    BN, Cin = featsT.shape
    g2 = 2 * wdT.shape[1]
    blk = 512
    grid = (BN // blk,)
    return pl.pallas_call(
        _table_body,
        grid=grid,
        in_specs=[
            pl.BlockSpec((blk, ptsT.shape[1]), lambda i: (i, 0)),
            pl.BlockSpec((blk, Cin), lambda i: (i, 0)),
            pl.BlockSpec(wdT.shape, lambda i: (0, 0)),
            pl.BlockSpec(wfT.shape, lambda i: (0, 0)),
            pl.BlockSpec(bf.shape, lambda i: (0, 0)),
        ],
        out_specs=pl.BlockSpec((blk, g2), lambda i: (i, 0)),
        out_shape=jax.ShapeDtypeStruct((BN, g2), jnp.float32),
    )(ptsT, featsT, wdT, wfT, bf)


def _gather_body(tbl_hbm, idx_hbm, bd_hbm, g1_hbm, g2_hbm, st_hbm,
                 idx_v, cent_v, rows_v, g1_v, g2_v, bd_v, stat_v,
                 gsem0, gsem1, wsem0, wsem1, *, n_chunks, cpw):
    cid = lax.axis_index("c")
    sid = lax.axis_index("s")
    wid = sid * 2 + cid

    epw = n_chunks * 128
    pltpu.sync_copy(idx_hbm.at[pl.ds(wid * n_chunks, n_chunks)], idx_v)
    pltpu.sync_copy(tbl_hbm.at[pl.ds(wid * cpw, cpw)], cent_v)
    pltpu.sync_copy(bd_hbm, bd_v)
    bd0 = bd_v[pl.ds(0, 16)]
    bd1 = bd_v[pl.ds(16, 16)]
    zero = jnp.zeros((16,), jnp.float32)
    gsems = (gsem0, gsem1)
    wsems = (wsem0, wsem1)

    def g_copy(c, s):
        return pltpu.make_async_copy(
            tbl_hbm.at[idx_v.at[c]], rows_v.at[s], gsems[s])

    def w_copies(c, s):
        n0 = wid * cpw + c * 8
        return (
            pltpu.make_async_copy(g1_v.at[s], g1_hbm.at[:, pl.ds(n0, 8), :],
                                  wsems[s]),
            pltpu.make_async_copy(g2_v.at[s], g2_hbm.at[:, pl.ds(n0, 8), :],
                                  wsems[s]),
        )

    g_copy(0, 0).start()
    g_copy(1, 1).start()

    def pair_body(j2, accs):
        for s in (0, 1):
            c = 2 * j2 + s
            g_copy(c, s).wait()

            @pl.when(c >= 2)
            def _():
                for d in w_copies(c - 2, s):
                    d.wait()

            rv = rows_v.at[s]
            g1v = g1_v.at[s]
            g2v = g2_v.at[s]

            def cen_body(c8, accs):
                nl = c * 8 + c8
                c0 = cent_v[nl, pl.ds(0, 16)]
                c1 = cent_v[nl, pl.ds(16, 16)]
                s1a, s1b, q1a, q1b, s2a, s2b, q2a, q2b = accs
                for e in range(16):
                    r = c8 * 16 + e
                    u0 = rv[r, pl.ds(0, 16)]
                    u1 = rv[r, pl.ds(16, 16)]
                    v0 = rv[r, pl.ds(32, 16)]
                    v1 = rv[r, pl.ds(48, 16)]
                    g1a = u0 - c0 + bd0
                    g1b = u1 - c1 + bd1
                    g1v[e, c8, pl.ds(0, 32)] = plsc.pack(
                        g1a, g1b, format=plsc.PackFormat.INTERLEAVED)
                    g2v[e, c8, pl.ds(0, 32)] = plsc.pack(
                        v0, v1, format=plsc.PackFormat.INTERLEAVED)
                    s1a = s1a + g1a
                    s1b = s1b + g1b
                    q1a = q1a + g1a * g1a
                    q1b = q1b + g1b * g1b
                    s2a = s2a + v0
                    s2b = s2b + v1
                    q2a = q2a + v0 * v0
                    q2b = q2b + v1 * v1
                return (s1a, s1b, q1a, q1b, s2a, s2b, q2a, q2b)

            accs = lax.fori_loop(0, 8, cen_body, accs)

            @pl.when(c + 2 < n_chunks)
            def _():
                g_copy(c + 2, s).start()

            for d in w_copies(c, s):
                d.start()
        return accs

    accs = lax.fori_loop(0, n_chunks // 2, pair_body, (zero,) * 8)
    for s in (0, 1):
        for d in w_copies(n_chunks - 2 + s, s):
            d.wait()
    for i in range(8):
        stat_v[i, pl.ds(0, 16)] = accs[i]
    pltpu.sync_copy(stat_v, st_hbm.at[wid])


def _gather_pass(tbl, idx2d, b_delta):
    BN = tbl.shape[0]
    n_rows = idx2d.shape[0]
    E = n_rows * 128
    n_chunks = n_rows // NW
    cpw = BN // NW
    mesh = plsc.VectorSubcoreMesh(core_axis_name="c", subcore_axis_name="s")
    body = functools.partial(_gather_body, n_chunks=n_chunks, cpw=cpw)
    f = pl.kernel(
        body,
        out_type=[
            jax.ShapeDtypeStruct((16, BN, 32), jnp.bfloat16),
            jax.ShapeDtypeStruct((16, BN, 32), jnp.bfloat16),
            jax.ShapeDtypeStruct((NW, 8, 16), jnp.float32),
        ],
        mesh=mesh,
        scratch_types=[
            pltpu.VMEM((n_chunks, 128), jnp.int32),
            pltpu.VMEM((cpw, 64), jnp.float32),
            pltpu.VMEM((2, 128, 64), jnp.float32),
            pltpu.VMEM((2, 16, 8, 32), jnp.bfloat16),
            pltpu.VMEM((2, 16, 8, 32), jnp.bfloat16),
            pltpu.VMEM((32,), jnp.float32),
            pltpu.VMEM((8, 16), jnp.float32),
            pltpu.SemaphoreType.DMA,
            pltpu.SemaphoreType.DMA,
            pltpu.SemaphoreType.DMA,
            pltpu.SemaphoreType.DMA,
        ],
        compiler_params=pltpu.CompilerParams(use_tc_tiling_on_sc=False,
                                             needs_layout_passes=False),
    )
    return f(tbl, idx2d, b_delta)


def _mix_body(g1_ref, g2_ref, a1, s1, a2, s2, wbd, bp, y_ref, st_ref, acc):
    i = pl.program_id(0)

    @pl.when(i == 0)
    def _():
        acc[...] = jnp.zeros_like(acc)

    x1 = g1_ref[...].astype(jnp.float32)
    x2 = g2_ref[...].astype(jnp.float32)
    w1 = jnp.maximum(x1 * a1[...] + s1[...], 0.0)
    w2 = jnp.maximum(x2 * a2[...] + s2[...], 0.0)
    z = w1 * w2
    y = jnp.dot(z, wbd[...], preferred_element_type=jnp.float32) + bp[...]
    y_ref[...] = y.astype(jnp.bfloat16)
    acc[0:1, :] += jnp.sum(y, axis=0, keepdims=True)
    acc[1:2, :] += jnp.sum(y * y, axis=0, keepdims=True)

    @pl.when(i == pl.num_programs(0) - 1)
    def _():
        st_ref[...] = acc[...]


def _mix_pass(G1r, G2r, a1, s1, a2, s2, wbd, bp):
    R = G1r.shape[0]
    blk = 2048
    grid = (R // blk,)
    vec = pl.BlockSpec((1, 128), lambda i: (0, 0))
    return pl.pallas_call(
        _mix_body,
        grid=grid,
        in_specs=[
            pl.BlockSpec((blk, 128), lambda i: (i, 0)),
            pl.BlockSpec((blk, 128), lambda i: (i, 0)),
            vec, vec, vec, vec,
            pl.BlockSpec((128, 128), lambda i: (0, 0)),
            vec,
        ],
        out_specs=[
            pl.BlockSpec((blk, 128), lambda i: (i, 0)),
            pl.BlockSpec((8, 128), lambda i: (0, 0)),
        ],
        out_shape=[
            jax.ShapeDtypeStruct((R, 128), jnp.bfloat16),
            jax.ShapeDtypeStruct((8, 128), jnp.float32),
        ],
        scratch_shapes=[pltpu.VMEM((8, 128), jnp.float32)],
    )(G1r, G2r, a1, s1, a2, s2, wbd, bp)


def _fold_body(y_ref, a3, s3, o_ref):
    j = pl.program_id(1)

    @pl.when(j == 0)
    def _():
        o_ref[...] = jnp.zeros_like(o_ref)

    y = y_ref[...].reshape(o_ref.shape).astype(jnp.float32)
    o_ref[...] += jnp.maximum(y * a3[...] + s3[...], 0.0)


def _fold_pass(y3k, a3, s3):
    K, R, _ = y3k.shape
    blk = 2048
    grid = (R // blk, K)
    vec = pl.BlockSpec((1, 128), lambda i, j: (0, 0))
    return pl.pallas_call(
        _fold_body,
        grid=grid,
        in_specs=[
            pl.BlockSpec((1, blk, 128), lambda i, j: (j, i, 0)),
            vec, vec,
        ],
        out_specs=pl.BlockSpec((blk, 128), lambda i, j: (i, 0)),
        out_shape=jax.ShapeDtypeStruct((R, 128), jnp.float32),
    )(y3k, a3, s3)


def _affine(sum_, sumsq, count, gamma, beta):
    mean = sum_ / count
    var = sumsq / count - mean * mean
    sc = gamma * lax.rsqrt(var + EPS)
    return sc, beta - sc * mean


def kernel(feats, pts, knn_idx,
           W_delta, b_delta, gamma_delta, beta_delta,
           W_feats, b_feats, gamma_feats, beta_feats,
           W_post, b_post, gamma_post, beta_post):
    B, Cin, N = feats.shape
    K = knn_idx.shape[-1]
    g = W_delta.shape[0]
    BN = B * N
    E = BN * K
    cnt = jnp.float32(E)
    perm = jnp.asarray(_PACK_PERM)

    ptsT = pts.transpose(0, 2, 1).reshape(BN, 3)
    ptsT = jnp.pad(ptsT, ((0, 0), (0, 5)))
    featsT = feats.transpose(0, 2, 1).reshape(BN, Cin)
    wdT = jnp.pad(W_delta, ((0, 0), (0, 5))).T
    wfT = W_feats.T
    bf = b_feats[None, :]

    idx_flat = (knn_idx.astype(jnp.int32)
                + (jnp.arange(B, dtype=jnp.int32) * N)[:, None, None])
    idx2d = idx_flat.reshape(E // 128, 128)

    tbl = _build_table(ptsT, featsT, wdT, wfT, bf)

    g1, g2, st = _gather_pass(tbl, idx2d, b_delta)

    parts = st.sum(axis=0)
    s1 = jnp.concatenate([parts[0], parts[1]])
    q1 = jnp.concatenate([parts[2], parts[3]])
    s2 = jnp.concatenate([parts[4], parts[5]])
    q2 = jnp.concatenate([parts[6], parts[7]])
    sc1, sh1 = _affine(s1, q1, cnt, gamma_delta, beta_delta)
    sc2, sh2 = _affine(s2, q2, cnt, gamma_feats, beta_feats)

    G1r = g1.reshape(E // 4, 128)
    G2r = g2.reshape(E // 4, 128)
    wbd = jnp.kron(jnp.eye(4, dtype=jnp.float32), W_post.T[perm, :])
    t4p = lambda x: jnp.tile(x[perm], 4)[None, :]
    t4 = lambda x: jnp.tile(x, 4)[None, :]
    y3r, st3 = _mix_pass(G1r, G2r, t4p(sc1), t4p(sh1), t4p(sc2), t4p(sh2),
                         wbd, t4(b_post))

    s3 = st3[0].reshape(4, g).sum(axis=0)
    q3 = st3[1].reshape(4, g).sum(axis=0)
    sc3, sh3 = _affine(s3, q3, cnt, gamma_post, beta_post)

    y3k = y3r.reshape(K, BN // 4, 128)
    out = _fold_pass(y3k, t4(sc3), t4(sh3))
    return out.reshape(B, N, g).transpose(0, 2, 1)


# speedup vs baseline: 1.6991x; 1.6991x over previous
import functools
import jax
import jax.numpy as jnp
from jax import lax
from jax.experimental import pallas as pl
from jax.experimental.pallas import tpu as pltpu
from jax.experimental.pallas import tpu_sc as plsc

EPS = 1e-5
NW = 32


def _table_body(p_ref, f_ref, wd_ref, wf_ref, bf_ref, o_ref):
    u = jnp.einsum('cn,oc->no', p_ref[0], wd_ref[...],
                   preferred_element_type=jnp.float32)
    v = jnp.einsum('cn,oc->no', f_ref[0], wf_ref[...],
                   preferred_element_type=jnp.float32)
    v = v + bf_ref[...]
    o_ref[...] = jnp.concatenate([u, v], axis=1)


def _build_table(pts, feats, W_delta, W_feats, bf):
    B, Cin, N = feats.shape
    g2 = 2 * W_delta.shape[0]
    blk = 512
    nb = N // blk
    grid = (B, nb)
    return pl.pallas_call(
        _table_body,
        grid=grid,
        in_specs=[
            pl.BlockSpec((1, pts.shape[1], blk), lambda b, i: (b, 0, i)),
            pl.BlockSpec((1, Cin, blk), lambda b, i: (b, 0, i)),
            pl.BlockSpec(W_delta.shape, lambda b, i: (0, 0)),
            pl.BlockSpec(W_feats.shape, lambda b, i: (0, 0)),
            pl.BlockSpec(bf.shape, lambda b, i: (0, 0)),
        ],
        out_specs=pl.BlockSpec((blk, g2), lambda b, i: (b * nb + i, 0)),
        out_shape=jax.ShapeDtypeStruct((B * N, g2), jnp.float32),
    )(pts, feats, W_delta, W_feats, bf)


def _gather_body(tbl_hbm, idx_hbm, bd_hbm, g1_hbm, g2_hbm, st_hbm,
                 idx_v, cent_v, rows_v, g1_v, g2_v, bd_v, stat_v,
                 gsem0, gsem1, wsem0, wsem1, *, n_chunks, cpw):
    cid = lax.axis_index("c")
    sid = lax.axis_index("s")
    wid = sid * 2 + cid

    pltpu.sync_copy(idx_hbm.at[pl.ds(wid * n_chunks, n_chunks)], idx_v)
    pltpu.sync_copy(tbl_hbm.at[pl.ds(wid * cpw, cpw)], cent_v)
    pltpu.sync_copy(bd_hbm, bd_v)
    bd0 = bd_v[pl.ds(0, 16)]
    bd1 = bd_v[pl.ds(16, 16)]
    zero = jnp.zeros((16,), jnp.float32)
    gsems = (gsem0, gsem1)
    wsems = (wsem0, wsem1)
    rpw = cpw // 4

    def g_copy(c, s):
        return pltpu.make_async_copy(
            tbl_hbm.at[idx_v.at[c]], rows_v.at[s], gsems[s])

    def w_copies(c, s):
        r0 = wid * rpw + c * 2
        return (
            pltpu.make_async_copy(g1_v.at[s], g1_hbm.at[:, pl.ds(r0, 2), :],
                                  wsems[s]),
            pltpu.make_async_copy(g2_v.at[s], g2_hbm.at[:, pl.ds(r0, 2), :],
                                  wsems[s]),
        )

    g_copy(0, 0).start()
    g_copy(1, 1).start()

    def pair_body(j2, accs):
        for s in (0, 1):
            c = 2 * j2 + s
            g_copy(c, s).wait()

            @pl.when(c >= 2)
            def _():
                for d in w_copies(c - 2, s):
                    d.wait()

            rv = rows_v.at[s]
            g1v = g1_v.at[s]
            g2v = g2_v.at[s]

            cents = []
            for c8 in range(8):
                nl = c * 8 + c8
                cents.append((cent_v[nl, pl.ds(0, 16)],
                              cent_v[nl, pl.ds(16, 16)]))

            def k_body(k, accs):
                s1a, s1b, q1a, q1b, s2a, s2b, q2a, q2b = accs
                for c8 in range(8):
                    r = c8 * 16 + k
                    rh = c8 // 4
                    lb = 32 * (c8 % 4)
                    c0, c1 = cents[c8]
                    u0 = rv[r, pl.ds(0, 16)]
                    u1 = rv[r, pl.ds(16, 16)]
                    v0 = rv[r, pl.ds(32, 16)]
                    v1 = rv[r, pl.ds(48, 16)]
                    g1a = u0 - c0 + bd0
                    g1b = u1 - c1 + bd1
                    g1v[k, rh, pl.ds(lb, 16)] = g1a
                    g1v[k, rh, pl.ds(lb + 16, 16)] = g1b
                    g2v[k, rh, pl.ds(lb, 16)] = v0
                    g2v[k, rh, pl.ds(lb + 16, 16)] = v1
                    s1a = s1a + g1a
                    s1b = s1b + g1b
                    q1a = q1a + g1a * g1a
                    q1b = q1b + g1b * g1b
                    s2a = s2a + v0
                    s2b = s2b + v1
                    q2a = q2a + v0 * v0
                    q2b = q2b + v1 * v1
                return (s1a, s1b, q1a, q1b, s2a, s2b, q2a, q2b)

            accs = lax.fori_loop(0, 16, k_body, accs)

            @pl.when(c + 2 < n_chunks)
            def _():
                g_copy(c + 2, s).start()

            for d in w_copies(c, s):
                d.start()
        return accs

    accs = lax.fori_loop(0, n_chunks // 2, pair_body, (zero,) * 8)
    for s in (0, 1):
        for d in w_copies(n_chunks - 2 + s, s):
            d.wait()
    for i in range(8):
        stat_v[i, pl.ds(0, 16)] = accs[i]
    pltpu.sync_copy(stat_v, st_hbm.at[wid])


def _gather_pass(tbl, idx2d, b_delta):
    BN = tbl.shape[0]
    n_rows = idx2d.shape[0]
    E = n_rows * 128
    K = 16
    n_chunks = n_rows // NW
    cpw = BN // NW
    mesh = plsc.VectorSubcoreMesh(core_axis_name="c", subcore_axis_name="s")
    body = functools.partial(_gather_body, n_chunks=n_chunks, cpw=cpw)
    f = pl.kernel(
        body,
        out_type=[
            jax.ShapeDtypeStruct((K, BN // 4, 128), jnp.float32),
            jax.ShapeDtypeStruct((K, BN // 4, 128), jnp.float32),
            jax.ShapeDtypeStruct((NW, 8, 16), jnp.float32),
        ],
        mesh=mesh,
        scratch_types=[
            pltpu.VMEM((n_chunks, 128), jnp.int32),
            pltpu.VMEM((cpw, 64), jnp.float32),
            pltpu.VMEM((2, 128, 64), jnp.float32),
            pltpu.VMEM((2, K, 2, 128), jnp.float32),
            pltpu.VMEM((2, K, 2, 128), jnp.float32),
            pltpu.VMEM((32,), jnp.float32),
            pltpu.VMEM((8, 16), jnp.float32),
            pltpu.SemaphoreType.DMA,
            pltpu.SemaphoreType.DMA,
            pltpu.SemaphoreType.DMA,
            pltpu.SemaphoreType.DMA,
        ],
        compiler_params=pltpu.CompilerParams(use_tc_tiling_on_sc=False,
                                             needs_layout_passes=False),
    )
    return f(tbl, idx2d, b_delta)


def _mix_body(g1_ref, g2_ref, a1, s1, a2, s2, wbd, bp, y_ref, st_ref, acc):
    i = pl.program_id(0)

    @pl.when(i == 0)
    def _():
        acc[...] = jnp.zeros_like(acc)

    w1 = jnp.maximum(g1_ref[...] * a1[...] + s1[...], 0.0)
    w2 = jnp.maximum(g2_ref[...] * a2[...] + s2[...], 0.0)
    z = w1 * w2
    y = jnp.dot(z, wbd[...], preferred_element_type=jnp.float32) + bp[...]
    y_ref[...] = y
    acc[0:1, :] += jnp.sum(y, axis=0, keepdims=True)
    acc[1:2, :] += jnp.sum(y * y, axis=0, keepdims=True)

    @pl.when(i == pl.num_programs(0) - 1)
    def _():
        st_ref[...] = acc[...]


def _mix_pass(G1r, G2r, a1, s1, a2, s2, wbd, bp):
    R = G1r.shape[0]
    blk = 2048
    grid = (R // blk,)
    vec = pl.BlockSpec((1, 128), lambda i: (0, 0))
    return pl.pallas_call(
        _mix_body,
        grid=grid,
        in_specs=[
            pl.BlockSpec((blk, 128), lambda i: (i, 0)),
            pl.BlockSpec((blk, 128), lambda i: (i, 0)),
            vec, vec, vec, vec,
            pl.BlockSpec((128, 128), lambda i: (0, 0)),
            vec,
        ],
        out_specs=[
            pl.BlockSpec((blk, 128), lambda i: (i, 0)),
            pl.BlockSpec((8, 128), lambda i: (0, 0)),
        ],
        out_shape=[
            jax.ShapeDtypeStruct((R, 128), jnp.float32),
            jax.ShapeDtypeStruct((8, 128), jnp.float32),
        ],
        scratch_shapes=[pltpu.VMEM((8, 128), jnp.float32)],
    )(G1r, G2r, a1, s1, a2, s2, wbd, bp)


def _fold_body(y_ref, a3, s3, o_ref):
    j = pl.program_id(1)

    @pl.when(j == 0)
    def _():
        o_ref[...] = jnp.zeros_like(o_ref)

    y = y_ref[...].reshape(o_ref.shape)
    o_ref[...] += jnp.maximum(y * a3[...] + s3[...], 0.0)


def _fold_pass(y3k, a3, s3):
    K, R, _ = y3k.shape
    blk = 2048
    grid = (R // blk, K)
    vec = pl.BlockSpec((1, 128), lambda i, j: (0, 0))
    return pl.pallas_call(
        _fold_body,
        grid=grid,
        in_specs=[
            pl.BlockSpec((1, blk, 128), lambda i, j: (j, i, 0)),
            vec, vec,
        ],
        out_specs=pl.BlockSpec((blk, 128), lambda i, j: (i, 0)),
        out_shape=jax.ShapeDtypeStruct((R, 128), jnp.float32),
    )(y3k, a3, s3)


def _affine(sum_, sumsq, count, gamma, beta):
    mean = sum_ / count
    var = sumsq / count - mean * mean
    sc = gamma * lax.rsqrt(var + EPS)
    return sc, beta - sc * mean


def kernel(feats, pts, knn_idx,
           W_delta, b_delta, gamma_delta, beta_delta,
           W_feats, b_feats, gamma_feats, beta_feats,
           W_post, b_post, gamma_post, beta_post):
    B, Cin, N = feats.shape
    K = knn_idx.shape[-1]
    g = W_delta.shape[0]
    BN = B * N
    E = BN * K
    cnt = jnp.float32(E)

    idx_flat = (knn_idx.astype(jnp.int32)
                + (jnp.arange(B, dtype=jnp.int32) * N)[:, None, None])
    idx2d = idx_flat.reshape(E // 128, 128)

    tbl = _build_table(pts, feats, W_delta, W_feats, b_feats[None, :])

    g1, g2, st = _gather_pass(tbl, idx2d, b_delta)

    parts = st.sum(axis=0)
    s1 = jnp.concatenate([parts[0], parts[1]])
    q1 = jnp.concatenate([parts[2], parts[3]])
    s2 = jnp.concatenate([parts[4], parts[5]])
    q2 = jnp.concatenate([parts[6], parts[7]])
    sc1, sh1 = _affine(s1, q1, cnt, gamma_delta, beta_delta)
    sc2, sh2 = _affine(s2, q2, cnt, gamma_feats, beta_feats)

    G1r = g1.reshape(E // 4, 128)
    G2r = g2.reshape(E // 4, 128)
    wbd = jnp.kron(jnp.eye(4, dtype=jnp.float32), W_post.T)
    t4 = lambda x: jnp.tile(x, 4)[None, :]
    y3r, st3 = _mix_pass(G1r, G2r, t4(sc1), t4(sh1), t4(sc2), t4(sh2),
                         wbd, t4(b_post))

    s3 = st3[0].reshape(4, g).sum(axis=0)
    q3 = st3[1].reshape(4, g).sum(axis=0)
    sc3, sh3 = _affine(s3, q3, cnt, gamma_post, beta_post)

    y3k = y3r.reshape(K, BN // 4, 128)
    out = _fold_pass(y3k, t4(sc3), t4(sh3))
    return out.reshape(B, N, g).transpose(0, 2, 1)


# speedup vs baseline: 2.2189x; 1.3059x over previous
import functools
import jax
import jax.numpy as jnp
from jax import lax
from jax.experimental import pallas as pl
from jax.experimental.pallas import tpu as pltpu
from jax.experimental.pallas import tpu_sc as plsc

EPS = 1e-5
NW = 32


def _table_body(p_ref, f_ref, wd_ref, wf_ref, bf_ref, o_ref):
    u = jnp.einsum('cn,oc->no', p_ref[0], wd_ref[...],
                   preferred_element_type=jnp.float32)
    v = jnp.einsum('cn,oc->no', f_ref[0], wf_ref[...],
                   preferred_element_type=jnp.float32)
    v = v + bf_ref[...]
    o_ref[...] = jnp.concatenate([u, v], axis=1)


def _build_table(pts, feats, W_delta, W_feats, bf):
    B, Cin, N = feats.shape
    g2 = 2 * W_delta.shape[0]
    blk = 1024
    nb = N // blk
    grid = (B, nb)
    return pl.pallas_call(
        _table_body,
        grid=grid,
        in_specs=[
            pl.BlockSpec((1, pts.shape[1], blk), lambda b, i: (b, 0, i)),
            pl.BlockSpec((1, Cin, blk), lambda b, i: (b, 0, i)),
            pl.BlockSpec(W_delta.shape, lambda b, i: (0, 0)),
            pl.BlockSpec(W_feats.shape, lambda b, i: (0, 0)),
            pl.BlockSpec(bf.shape, lambda b, i: (0, 0)),
        ],
        out_specs=pl.BlockSpec((blk, g2), lambda b, i: (b * nb + i, 0)),
        out_shape=jax.ShapeDtypeStruct((B * N, g2), jnp.float32),
    )(pts, feats, W_delta, W_feats, bf)


def _gather_body(tbl_hbm, idx_hbm, bd_hbm, g1_hbm, g2_hbm, st_hbm,
                 idx_v, cent_v, rows_v, g1_v, g2_v, bd_v, stat_v,
                 gsem0, gsem1, wsem0, wsem1, *, n_chunks, cpw):
    cid = lax.axis_index("c")
    sid = lax.axis_index("s")
    wid = sid * 2 + cid

    pltpu.sync_copy(idx_hbm.at[pl.ds(wid * n_chunks, n_chunks)], idx_v)
    pltpu.sync_copy(tbl_hbm.at[pl.ds(wid * cpw, cpw)], cent_v)
    pltpu.sync_copy(bd_hbm, bd_v)
    bd0 = bd_v[pl.ds(0, 16)]
    bd1 = bd_v[pl.ds(16, 16)]
    zero = jnp.zeros((16,), jnp.float32)
    gsems = (gsem0, gsem1)
    wsems = (wsem0, wsem1)
    rpw = cpw // 8

    def g_copy(c, s):
        return pltpu.make_async_copy(
            tbl_hbm.at[idx_v.at[c]], rows_v.at[s], gsems[s])

    def w_copies(c, s):
        m0 = wid * rpw + c
        return (
            pltpu.make_async_copy(g1_v.at[s], g1_hbm.at[:, pl.ds(m0, 1), :],
                                  wsems[s]),
            pltpu.make_async_copy(g2_v.at[s], g2_hbm.at[:, pl.ds(m0, 1), :],
                                  wsems[s]),
        )

    g_copy(0, 0).start()
    g_copy(1, 1).start()

    def pair_body(j2, accs):
        for s in (0, 1):
            c = 2 * j2 + s
            g_copy(c, s).wait()

            @pl.when(c >= 2)
            def _():
                for d in w_copies(c - 2, s):
                    d.wait()

            rv = rows_v.at[s]
            g1v = g1_v.at[s]
            g2v = g2_v.at[s]

            cents = []
            for c8 in range(8):
                nl = c * 8 + c8
                cents.append((cent_v[nl, pl.ds(0, 16)],
                              cent_v[nl, pl.ds(16, 16)]))

            def k_body(kk, accs):
                s1a, s1b, q1a, q1b, s2a, s2b, q2a, q2b = accs
                for dk in range(4):
                    k = kk * 4 + dk
                    e1a, e1b, e2a, e2b = [], [], [], []
                    for c8 in range(8):
                        r = c8 * 16 + k
                        c0, c1 = cents[c8]
                        u0 = rv[r, pl.ds(0, 16)]
                        u1 = rv[r, pl.ds(16, 16)]
                        v0 = rv[r, pl.ds(32, 16)]
                        v1 = rv[r, pl.ds(48, 16)]
                        g1a = u0 - c0 + bd0
                        g1b = u1 - c1 + bd1
                        e1a.append(g1a)
                        e1b.append(g1b)
                        e2a.append(v0)
                        e2b.append(v1)
                        s1a = s1a + g1a
                        s1b = s1b + g1b
                        q1a = q1a + g1a * g1a
                        q1b = q1b + g1b * g1b
                        s2a = s2a + v0
                        s2b = s2b + v1
                        q2a = q2a + v0 * v0
                        q2b = q2b + v1 * v1
                    for q in range(4):
                        for dst, lo, hi in ((g1v, e1a, e1b), (g2v, e2a, e2b)):
                            wlo = plsc.bitcast(plsc.pack(
                                lo[q], lo[q + 4],
                                format=plsc.PackFormat.INTERLEAVED), jnp.int32)
                            whi = plsc.bitcast(plsc.pack(
                                hi[q], hi[q + 4],
                                format=plsc.PackFormat.INTERLEAVED), jnp.int32)
                            dst[k, 0, pl.ds(32 * q, 16)] = wlo
                            dst[k, 0, pl.ds(32 * q + 16, 16)] = whi
                return (s1a, s1b, q1a, q1b, s2a, s2b, q2a, q2b)

            accs = lax.fori_loop(0, 4, k_body, accs)

            @pl.when(c + 2 < n_chunks)
            def _():
                g_copy(c + 2, s).start()

            for d in w_copies(c, s):
                d.start()
        return accs

    accs = lax.fori_loop(0, n_chunks // 2, pair_body, (zero,) * 8)
    for s in (0, 1):
        for d in w_copies(n_chunks - 2 + s, s):
            d.wait()
    for i in range(8):
        stat_v[i, pl.ds(0, 16)] = accs[i]
    pltpu.sync_copy(stat_v, st_hbm.at[wid])


def _gather_pass(tbl, idx2d, b_delta):
    BN = tbl.shape[0]
    n_rows = idx2d.shape[0]
    E = n_rows * 128
    K = 16
    n_chunks = n_rows // NW
    cpw = BN // NW
    mesh = plsc.VectorSubcoreMesh(core_axis_name="c", subcore_axis_name="s")
    body = functools.partial(_gather_body, n_chunks=n_chunks, cpw=cpw)
    f = pl.kernel(
        body,
        out_type=[
            jax.ShapeDtypeStruct((K, BN // 8, 128), jnp.int32),
            jax.ShapeDtypeStruct((K, BN // 8, 128), jnp.int32),
            jax.ShapeDtypeStruct((NW, 8, 16), jnp.float32),
        ],
        mesh=mesh,
        scratch_types=[
            pltpu.VMEM((n_chunks, 128), jnp.int32),
            pltpu.VMEM((cpw, 64), jnp.float32),
            pltpu.VMEM((2, 128, 64), jnp.float32),
            pltpu.VMEM((2, K, 1, 128), jnp.int32),
            pltpu.VMEM((2, K, 1, 128), jnp.int32),
            pltpu.VMEM((32,), jnp.float32),
            pltpu.VMEM((8, 16), jnp.float32),
            pltpu.SemaphoreType.DMA,
            pltpu.SemaphoreType.DMA,
            pltpu.SemaphoreType.DMA,
            pltpu.SemaphoreType.DMA,
        ],
        compiler_params=pltpu.CompilerParams(use_tc_tiling_on_sc=False,
                                             needs_layout_passes=False),
    )
    return f(tbl, idx2d, b_delta)


def _unpack2(x):
    xa = lax.bitcast_convert_type(lax.shift_left(x, jnp.int32(16)),
                                  jnp.float32)
    xb = lax.bitcast_convert_type(lax.bitwise_and(x, jnp.int32(-65536)),
                                  jnp.float32)
    return xa, xb


def _pack2(ya, yb):
    ua = lax.shift_right_logical(
        lax.bitcast_convert_type(ya.astype(jnp.bfloat16).astype(jnp.float32),
                                 jnp.uint32), jnp.uint32(16))
    ub = lax.bitwise_and(
        lax.bitcast_convert_type(yb.astype(jnp.bfloat16).astype(jnp.float32),
                                 jnp.uint32), jnp.uint32(0xFFFF0000))
    return lax.bitcast_convert_type(ua | ub, jnp.int32)


def _mix_body(g1_ref, g2_ref, a1, s1, a2, s2, wbd, bp, y_ref, st_ref, acc):
    i = pl.program_id(0)

    @pl.when(i == 0)
    def _():
        acc[...] = jnp.zeros_like(acc)

    x1a, x1b = _unpack2(g1_ref[...])
    x2a, x2b = _unpack2(g2_ref[...])
    za = (jnp.maximum(x1a * a1[...] + s1[...], 0.0)
          * jnp.maximum(x2a * a2[...] + s2[...], 0.0))
    zb = (jnp.maximum(x1b * a1[...] + s1[...], 0.0)
          * jnp.maximum(x2b * a2[...] + s2[...], 0.0))
    ya = jnp.dot(za, wbd[...], preferred_element_type=jnp.float32) + bp[...]
    yb = jnp.dot(zb, wbd[...], preferred_element_type=jnp.float32) + bp[...]
    y_ref[...] = _pack2(ya, yb)
    acc[0:1, :] += (jnp.sum(ya, axis=0, keepdims=True)
                    + jnp.sum(yb, axis=0, keepdims=True))
    acc[1:2, :] += (jnp.sum(ya * ya, axis=0, keepdims=True)
                    + jnp.sum(yb * yb, axis=0, keepdims=True))

    @pl.when(i == pl.num_programs(0) - 1)
    def _():
        st_ref[...] = acc[...]


def _mix_pass(G1r, G2r, a1, s1, a2, s2, wbd, bp):
    R = G1r.shape[0]
    blk = 2048
    grid = (R // blk,)
    vec = pl.BlockSpec((1, 128), lambda i: (0, 0))
    return pl.pallas_call(
        _mix_body,
        grid=grid,
        in_specs=[
            pl.BlockSpec((blk, 128), lambda i: (i, 0)),
            pl.BlockSpec((blk, 128), lambda i: (i, 0)),
            vec, vec, vec, vec,
            pl.BlockSpec((128, 128), lambda i: (0, 0)),
            vec,
        ],
        out_specs=[
            pl.BlockSpec((blk, 128), lambda i: (i, 0)),
            pl.BlockSpec((8, 128), lambda i: (0, 0)),
        ],
        out_shape=[
            jax.ShapeDtypeStruct((R, 128), jnp.int32),
            jax.ShapeDtypeStruct((8, 128), jnp.float32),
        ],
        scratch_shapes=[pltpu.VMEM((8, 128), jnp.float32)],
    )(G1r, G2r, a1, s1, a2, s2, wbd, bp)


def _fold_body(y_ref, a3, s3, o_ref):
    j = pl.program_id(1)

    @pl.when(j == 0)
    def _():
        o_ref[...] = jnp.zeros_like(o_ref)

    blk = o_ref.shape[0]
    ya, yb = _unpack2(y_ref[...].reshape(blk, 128))
    o_ref[:, 0:128] += jnp.maximum(ya * a3[...] + s3[...], 0.0)
    o_ref[:, 128:256] += jnp.maximum(yb * a3[...] + s3[...], 0.0)


def _fold_pass(y3k, a3, s3):
    K, R, _ = y3k.shape
    blk = 2048
    grid = (R // blk, K)
    vec = pl.BlockSpec((1, 128), lambda i, j: (0, 0))
    return pl.pallas_call(
        _fold_body,
        grid=grid,
        in_specs=[
            pl.BlockSpec((1, blk, 128), lambda i, j: (j, i, 0)),
            vec, vec,
        ],
        out_specs=pl.BlockSpec((blk, 256), lambda i, j: (i, 0)),
        out_shape=jax.ShapeDtypeStruct((R, 256), jnp.float32),
    )(y3k, a3, s3)


def _affine(sum_, sumsq, count, gamma, beta):
    mean = sum_ / count
    var = sumsq / count - mean * mean
    sc = gamma * lax.rsqrt(var + EPS)
    return sc, beta - sc * mean


def kernel(feats, pts, knn_idx,
           W_delta, b_delta, gamma_delta, beta_delta,
           W_feats, b_feats, gamma_feats, beta_feats,
           W_post, b_post, gamma_post, beta_post):
    B, Cin, N = feats.shape
    K = knn_idx.shape[-1]
    g = W_delta.shape[0]
    BN = B * N
    E = BN * K
    cnt = jnp.float32(E)

    idx_flat = (knn_idx.astype(jnp.int32)
                + (jnp.arange(B, dtype=jnp.int32) * N)[:, None, None])
    idx2d = idx_flat.reshape(E // 128, 128)

    tbl = _build_table(pts, feats, W_delta, W_feats, b_feats[None, :])

    g1, g2, st = _gather_pass(tbl, idx2d, b_delta)

    parts = st.sum(axis=0)
    s1 = jnp.concatenate([parts[0], parts[1]])
    q1 = jnp.concatenate([parts[2], parts[3]])
    s2 = jnp.concatenate([parts[4], parts[5]])
    q2 = jnp.concatenate([parts[6], parts[7]])
    sc1, sh1 = _affine(s1, q1, cnt, gamma_delta, beta_delta)
    sc2, sh2 = _affine(s2, q2, cnt, gamma_feats, beta_feats)

    G1r = g1.reshape(E // 8, 128)
    G2r = g2.reshape(E // 8, 128)
    wbd = jnp.kron(jnp.eye(4, dtype=jnp.float32), W_post.T)
    t4 = lambda x: jnp.tile(x, 4)[None, :]
    y3r, st3 = _mix_pass(G1r, G2r, t4(sc1), t4(sh1), t4(sc2), t4(sh2),
                         wbd, t4(b_post))

    s3 = st3[0].reshape(4, g).sum(axis=0)
    q3 = st3[1].reshape(4, g).sum(axis=0)
    sc3, sh3 = _affine(s3, q3, cnt, gamma_post, beta_post)

    y3k = y3r.reshape(K, BN // 8, 128)
    out = _fold_pass(y3k, t4(sc3), t4(sh3))
    return out.reshape(B, N, g).transpose(0, 2, 1)


# speedup vs baseline: 2.2800x; 1.0275x over previous
import functools
import jax
import jax.numpy as jnp
from jax import lax
from jax.experimental import pallas as pl
from jax.experimental.pallas import tpu as pltpu
from jax.experimental.pallas import tpu_sc as plsc

EPS = 1e-5
NW = 32


def _table_body(p_ref, f_ref, wd_ref, wf_ref, bf_ref, o_ref):
    u = jnp.einsum('cn,oc->no', p_ref[0], wd_ref[...],
                   preferred_element_type=jnp.float32)
    v = jnp.einsum('cn,oc->no', f_ref[...], wf_ref[...],
                   preferred_element_type=jnp.float32)
    v = v + bf_ref[...]
    uw = _pack2(u[:, 0:16], u[:, 16:32])
    vw = _pack2(v[:, 0:16], v[:, 16:32])
    o_ref[...] = jnp.concatenate([uw, vw], axis=1)


def _build_table(ptsr, featsr, W_delta, W_feats, bf, B):
    Cin, N = featsr.shape[0] // B, featsr.shape[1]
    g = W_delta.shape[0]
    blk = 1024
    nb = N // blk
    grid = (B, nb)
    return pl.pallas_call(
        _table_body,
        grid=grid,
        in_specs=[
            pl.BlockSpec((1, ptsr.shape[1], blk), lambda b, i: (b, 0, i)),
            pl.BlockSpec((Cin, blk), lambda b, i: (b, i)),
            pl.BlockSpec(W_delta.shape, lambda b, i: (0, 0)),
            pl.BlockSpec(W_feats.shape, lambda b, i: (0, 0)),
            pl.BlockSpec(bf.shape, lambda b, i: (0, 0)),
        ],
        out_specs=pl.BlockSpec((blk, g), lambda b, i: (b * nb + i, 0)),
        out_shape=jax.ShapeDtypeStruct((B * N, g), jnp.int32),
    )(ptsr, featsr, W_delta, W_feats, bf)


def _gather_body(tbl_hbm, idx_hbm, bd_hbm, g1_hbm, g2_hbm, st_hbm,
                 idx_v, cent_v, rows_v, g1_v, g2_v, bd_v, stat_v,
                 gsem0, gsem1, wsem0, wsem1, *, n_chunks, cpw):
    cid = lax.axis_index("c")
    sid = lax.axis_index("s")
    wid = sid * 2 + cid

    pltpu.sync_copy(idx_hbm.at[pl.ds(wid * n_chunks, n_chunks)], idx_v)
    pltpu.sync_copy(tbl_hbm.at[pl.ds(wid * cpw, cpw), pl.ds(0, 16)], cent_v)
    pltpu.sync_copy(bd_hbm, bd_v)
    bd0 = bd_v[pl.ds(0, 16)]
    bd1 = bd_v[pl.ds(16, 16)]
    zero = jnp.zeros((16,), jnp.float32)
    gsems = (gsem0, gsem1)
    wsems = (wsem0, wsem1)
    rpw = cpw // 8

    def g_copy(c, s):
        return pltpu.make_async_copy(
            tbl_hbm.at[idx_v.at[c]], rows_v.at[s], gsems[s])

    def w_copies(c, s):
        m0 = wid * rpw + c
        return (
            pltpu.make_async_copy(g1_v.at[s], g1_hbm.at[:, pl.ds(m0, 1), :],
                                  wsems[s]),
            pltpu.make_async_copy(g2_v.at[s], g2_hbm.at[:, pl.ds(m0, 1), :],
                                  wsems[s]),
        )

    g_copy(0, 0).start()
    g_copy(1, 1).start()

    def pair_body(j2, accs):
        for s in (0, 1):
            c = 2 * j2 + s
            g_copy(c, s).wait()

            @pl.when(c >= 2)
            def _():
                for d in w_copies(c - 2, s):
                    d.wait()

            rv = rows_v.at[s]
            g1v = g1_v.at[s]
            g2v = g2_v.at[s]

            cents = []
            for c8 in range(8):
                nl = c * 8 + c8
                cw = plsc.bitcast(cent_v[nl, pl.ds(0, 16)], jnp.bfloat16)
                cents.append(plsc.unpack(
                    cw, format=plsc.PackFormat.INTERLEAVED))

            def k_body(kk, accs):
                s1a, s1b, q1a, q1b, s2a, s2b, q2a, q2b = accs
                for dk in range(4):
                    k = kk * 4 + dk
                    e1a, e1b, e2a, e2b = [], [], [], []
                    for c8 in range(8):
                        r = c8 * 16 + k
                        c0, c1 = cents[c8]
                        uw = plsc.bitcast(rv[r, pl.ds(0, 16)], jnp.bfloat16)
                        vw = plsc.bitcast(rv[r, pl.ds(16, 16)], jnp.bfloat16)
                        u0, u1 = plsc.unpack(
                            uw, format=plsc.PackFormat.INTERLEAVED)
                        v0, v1 = plsc.unpack(
                            vw, format=plsc.PackFormat.INTERLEAVED)
                        g1a = u0 - c0 + bd0
                        g1b = u1 - c1 + bd1
                        e1a.append(g1a)
                        e1b.append(g1b)
                        e2a.append(v0)
                        e2b.append(v1)
                        s1a = s1a + g1a
                        s1b = s1b + g1b
                        q1a = q1a + g1a * g1a
                        q1b = q1b + g1b * g1b
                        s2a = s2a + v0
                        s2b = s2b + v1
                        q2a = q2a + v0 * v0
                        q2b = q2b + v1 * v1
                    for q in range(4):
                        for dst, lo, hi in ((g1v, e1a, e1b), (g2v, e2a, e2b)):
                            wlo = plsc.bitcast(plsc.pack(
                                lo[q], lo[q + 4],
                                format=plsc.PackFormat.INTERLEAVED), jnp.int32)
                            whi = plsc.bitcast(plsc.pack(
                                hi[q], hi[q + 4],
                                format=plsc.PackFormat.INTERLEAVED), jnp.int32)
                            dst[k, 0, pl.ds(32 * q, 16)] = wlo
                            dst[k, 0, pl.ds(32 * q + 16, 16)] = whi
                return (s1a, s1b, q1a, q1b, s2a, s2b, q2a, q2b)

            accs = lax.fori_loop(0, 4, k_body, accs)

            @pl.when(c + 2 < n_chunks)
            def _():
                g_copy(c + 2, s).start()

            for d in w_copies(c, s):
                d.start()
        return accs

    accs = lax.fori_loop(0, n_chunks // 2, pair_body, (zero,) * 8)
    for s in (0, 1):
        for d in w_copies(n_chunks - 2 + s, s):
            d.wait()
    for i in range(8):
        stat_v[i, pl.ds(0, 16)] = accs[i]
    pltpu.sync_copy(stat_v, st_hbm.at[wid])


def _gather_pass(tbl, idx2d, b_delta):
    BN = tbl.shape[0]
    n_rows = idx2d.shape[0]
    E = n_rows * 128
    K = 16
    n_chunks = n_rows // NW
    cpw = BN // NW
    mesh = plsc.VectorSubcoreMesh(core_axis_name="c", subcore_axis_name="s")
    body = functools.partial(_gather_body, n_chunks=n_chunks, cpw=cpw)
    f = pl.kernel(
        body,
        out_type=[
            jax.ShapeDtypeStruct((K, BN // 8, 128), jnp.int32),
            jax.ShapeDtypeStruct((K, BN // 8, 128), jnp.int32),
            jax.ShapeDtypeStruct((NW, 8, 16), jnp.float32),
        ],
        mesh=mesh,
        scratch_types=[
            pltpu.VMEM((n_chunks, 128), jnp.int32),
            pltpu.VMEM((cpw, 16), jnp.int32),
            pltpu.VMEM((2, 128, 32), jnp.int32),
            pltpu.VMEM((2, K, 1, 128), jnp.int32),
            pltpu.VMEM((2, K, 1, 128), jnp.int32),
            pltpu.VMEM((32,), jnp.float32),
            pltpu.VMEM((8, 16), jnp.float32),
            pltpu.SemaphoreType.DMA,
            pltpu.SemaphoreType.DMA,
            pltpu.SemaphoreType.DMA,
            pltpu.SemaphoreType.DMA,
        ],
        compiler_params=pltpu.CompilerParams(use_tc_tiling_on_sc=False,
                                             needs_layout_passes=False),
    )
    return f(tbl, idx2d, b_delta)


def _unpack2(x):
    xa = lax.bitcast_convert_type(lax.shift_left(x, jnp.int32(16)),
                                  jnp.float32)
    xb = lax.bitcast_convert_type(lax.bitwise_and(x, jnp.int32(-65536)),
                                  jnp.float32)
    return xa, xb


def _pack2(ya, yb):
    ua = lax.shift_right_logical(
        lax.bitcast_convert_type(ya.astype(jnp.bfloat16).astype(jnp.float32),
                                 jnp.uint32), jnp.uint32(16))
    ub = lax.bitwise_and(
        lax.bitcast_convert_type(yb.astype(jnp.bfloat16).astype(jnp.float32),
                                 jnp.uint32), jnp.uint32(0xFFFF0000))
    return lax.bitcast_convert_type(ua | ub, jnp.int32)


def _mix_body(g1_ref, g2_ref, a1, s1, a2, s2, wbd, bp, y_ref, st_ref, acc):
    i = pl.program_id(0)

    @pl.when(i == 0)
    def _():
        acc[...] = jnp.zeros_like(acc)

    x1a, x1b = _unpack2(g1_ref[...])
    x2a, x2b = _unpack2(g2_ref[...])
    za = (jnp.maximum(x1a * a1[...] + s1[...], 0.0)
          * jnp.maximum(x2a * a2[...] + s2[...], 0.0))
    zb = (jnp.maximum(x1b * a1[...] + s1[...], 0.0)
          * jnp.maximum(x2b * a2[...] + s2[...], 0.0))
    ya = jnp.dot(za, wbd[...], preferred_element_type=jnp.float32) + bp[...]
    yb = jnp.dot(zb, wbd[...], preferred_element_type=jnp.float32) + bp[...]
    y_ref[...] = _pack2(ya, yb)
    acc[0:1, :] += (jnp.sum(ya, axis=0, keepdims=True)
                    + jnp.sum(yb, axis=0, keepdims=True))
    acc[1:2, :] += (jnp.sum(ya * ya, axis=0, keepdims=True)
                    + jnp.sum(yb * yb, axis=0, keepdims=True))

    @pl.when(i == pl.num_programs(0) - 1)
    def _():
        st_ref[...] = acc[...]


def _mix_pass(G1r, G2r, a1, s1, a2, s2, wbd, bp):
    R = G1r.shape[0]
    blk = 2048
    grid = (R // blk,)
    vec = pl.BlockSpec((1, 128), lambda i: (0, 0))
    return pl.pallas_call(
        _mix_body,
        grid=grid,
        in_specs=[
            pl.BlockSpec((blk, 128), lambda i: (i, 0)),
            pl.BlockSpec((blk, 128), lambda i: (i, 0)),
            vec, vec, vec, vec,
            pl.BlockSpec((128, 128), lambda i: (0, 0)),
            vec,
        ],
        out_specs=[
            pl.BlockSpec((blk, 128), lambda i: (i, 0)),
            pl.BlockSpec((8, 128), lambda i: (0, 0)),
        ],
        out_shape=[
            jax.ShapeDtypeStruct((R, 128), jnp.int32),
            jax.ShapeDtypeStruct((8, 128), jnp.float32),
        ],
        scratch_shapes=[pltpu.VMEM((8, 128), jnp.float32)],
    )(G1r, G2r, a1, s1, a2, s2, wbd, bp)


def _fold_body(y_ref, a3, s3, o_ref):
    j = pl.program_id(1)

    @pl.when(j == 0)
    def _():
        o_ref[...] = jnp.zeros_like(o_ref)

    blk = o_ref.shape[0]
    ya, yb = _unpack2(y_ref[...].reshape(blk, 128))
    o_ref[:, 0:128] += jnp.maximum(ya * a3[...] + s3[...], 0.0)
    o_ref[:, 128:256] += jnp.maximum(yb * a3[...] + s3[...], 0.0)


def _fold_pass(y3k, a3, s3):
    K, R, _ = y3k.shape
    blk = 2048
    grid = (R // blk, K)
    vec = pl.BlockSpec((1, 128), lambda i, j: (0, 0))
    return pl.pallas_call(
        _fold_body,
        grid=grid,
        in_specs=[
            pl.BlockSpec((1, blk, 128), lambda i, j: (j, i, 0)),
            vec, vec,
        ],
        out_specs=pl.BlockSpec((blk, 256), lambda i, j: (i, 0)),
        out_shape=jax.ShapeDtypeStruct((R, 256), jnp.float32),
    )(y3k, a3, s3)


def _affine(sum_, sumsq, count, gamma, beta):
    mean = sum_ / count
    var = sumsq / count - mean * mean
    sc = gamma * lax.rsqrt(var + EPS)
    return sc, beta - sc * mean


def kernel(feats, pts, knn_idx,
           W_delta, b_delta, gamma_delta, beta_delta,
           W_feats, b_feats, gamma_feats, beta_feats,
           W_post, b_post, gamma_post, beta_post):
    B, Cin, N = feats.shape
    K = knn_idx.shape[-1]
    g = W_delta.shape[0]
    BN = B * N
    E = BN * K
    cnt = jnp.float32(E)

    idx_flat = (knn_idx.astype(jnp.int32)
                + (jnp.arange(B, dtype=jnp.int32) * N)[:, None, None])
    idx2d = idx_flat.reshape(E // 128, 128)

    tbl = _build_table(pts, feats.reshape(B * Cin, N),
                       W_delta, W_feats, b_feats[None, :], B)

    g1, g2, st = _gather_pass(tbl, idx2d, b_delta)

    parts = st.sum(axis=0)
    s1 = jnp.concatenate([parts[0], parts[1]])
    q1 = jnp.concatenate([parts[2], parts[3]])
    s2 = jnp.concatenate([parts[4], parts[5]])
    q2 = jnp.concatenate([parts[6], parts[7]])
    sc1, sh1 = _affine(s1, q1, cnt, gamma_delta, beta_delta)
    sc2, sh2 = _affine(s2, q2, cnt, gamma_feats, beta_feats)

    G1r = g1.reshape(E // 8, 128)
    G2r = g2.reshape(E // 8, 128)
    wbd = jnp.kron(jnp.eye(4, dtype=jnp.float32), W_post.T)
    t4 = lambda x: jnp.tile(x, 4)[None, :]
    y3r, st3 = _mix_pass(G1r, G2r, t4(sc1), t4(sh1), t4(sc2), t4(sh2),
                         wbd, t4(b_post))

    s3 = st3[0].reshape(4, g).sum(axis=0)
    q3 = st3[1].reshape(4, g).sum(axis=0)
    sc3, sh3 = _affine(s3, q3, cnt, gamma_post, beta_post)

    y3k = y3r.reshape(K, BN // 8, 128)
    out = _fold_pass(y3k, t4(sc3), t4(sh3))
    return out.reshape(B, N, g).transpose(0, 2, 1)
